# speedup vs baseline: 6.9663x; 6.9663x over previous
import functools

import jax
import jax.numpy as jnp
from jax import lax
from jax.experimental import pallas as pl
from jax.experimental.pallas import tpu as pltpu
from jax.experimental.pallas import tpu_sc as plsc

N_NODES = 10000
D_FEAT = 128
N_EDGES = 320000

NC = 2
NS = 16
NW = NC * NS

EDGE_B = 128
N_BATCH = N_EDGES // EDGE_B
BATCH_PER_W = -(-N_BATCH // NW)

ROW_CHUNK = 200
N_CHUNK = N_NODES // ROW_CHUNK
CHUNK_PER_S = -(-N_CHUNK // NS)


def _sc_partial(x, edge_index):
    mesh = plsc.VectorSubcoreMesh(core_axis_name="c", subcore_axis_name="s")

    @functools.partial(
        pl.kernel,
        out_type=jax.ShapeDtypeStruct((NC, N_NODES, D_FEAT), jnp.float32),
        mesh=mesh,
        scratch_types=dict(
            zbuf=pltpu.VMEM((ROW_CHUNK, D_FEAT), jnp.float32),
            sidx=pltpu.VMEM((EDGE_B,), jnp.int32),
            didx=pltpu.VMEM((EDGE_B,), jnp.int32),
            rows=pltpu.VMEM((EDGE_B, D_FEAT), jnp.float32),
            acc=pltpu.VMEM_SHARED((N_NODES, D_FEAT), jnp.float32),
            sem=pltpu.SemaphoreType.DMA,
        ),
    )
    def kern(x_hbm, ei_hbm, part_hbm, *, zbuf, sidx, didx, rows, acc, sem):
        c = lax.axis_index("c")
        s = lax.axis_index("s")
        w = c * NS + s

        zero = jnp.zeros((16,), jnp.float32)

        def zrow(r, _):
            def zcol(k, _):
                zbuf[r, pl.ds(k * 16, 16)] = zero
                return 0
            return lax.fori_loop(0, D_FEAT // 16, zcol, 0)

        lax.fori_loop(0, ROW_CHUNK, zrow, 0)

        def zchunk(i, _):
            ch = s + i * NS

            @pl.when(ch < N_CHUNK)
            def _():
                pltpu.sync_copy(zbuf, acc.at[pl.ds(ch * ROW_CHUNK, ROW_CHUNK), :])
            return 0

        lax.fori_loop(0, CHUNK_PER_S, zchunk, 0)
        plsc.subcore_barrier()

        def ebatch(i, _):
            bid = w + i * NW

            @pl.when(bid < N_BATCH)
            def _():
                base = bid * EDGE_B
                pltpu.sync_copy(ei_hbm.at[0, pl.ds(base, EDGE_B)], sidx)
                pltpu.sync_copy(ei_hbm.at[1, pl.ds(base, EDGE_B)], didx)
                pltpu.async_copy(x_hbm.at[sidx], rows, sem).wait()
                pltpu.sync_copy(rows, acc.at[didx], add=True)
            return 0

        lax.fori_loop(0, BATCH_PER_W, ebatch, 0)
        plsc.subcore_barrier()

        def wchunk(i, _):
            ch = s + i * NS

            @pl.when(ch < N_CHUNK)
            def _():
                r0 = ch * ROW_CHUNK
                pltpu.sync_copy(
                    acc.at[pl.ds(r0, ROW_CHUNK), :],
                    part_hbm.at[c, pl.ds(r0, ROW_CHUNK), :],
                )
            return 0

        lax.fori_loop(0, CHUNK_PER_S, wchunk, 0)

    return kern(x, edge_index)


def _combine(parts):
    blk = 400

    def body(p_ref, o_ref):
        o_ref[...] = p_ref[0] + p_ref[1]

    return pl.pallas_call(
        body,
        grid=(N_NODES // blk,),
        in_specs=[pl.BlockSpec((NC, blk, D_FEAT), lambda i: (0, i, 0))],
        out_specs=pl.BlockSpec((blk, D_FEAT), lambda i: (i, 0)),
        out_shape=jax.ShapeDtypeStruct((N_NODES, D_FEAT), jnp.float32),
    )(parts)


def kernel(x, edge_index):
    ei = edge_index.astype(jnp.int32)
    parts = _sc_partial(x, ei)
    return _combine(parts)
